# baseline (speedup 1.0000x reference)
import functools

import jax
import jax.numpy as jnp
from jax import lax
from jax.experimental import pallas as pl
from jax.experimental.pallas import tpu as pltpu
from jax.experimental.pallas import tpu_sc as plsc

NC = 2
NS = 16
NW = NC * NS
CHUNK = 128


@functools.partial(jax.jit, static_argnames=("batch", "seq", "emb"))
def _embedding_gather(idx3, table, *, batch, seq, emb):
    total = batch * seq
    bpw = total // NW
    n_chunks = bpw // CHUNK
    n_pairs = n_chunks // 2 - 1

    mesh = plsc.VectorSubcoreMesh(core_axis_name="c", subcore_axis_name="s")

    @functools.partial(
        pl.kernel,
        mesh=mesh,
        compiler_params=pltpu.CompilerParams(use_tc_tiling_on_sc=False),
        out_type=jax.ShapeDtypeStruct((total, emb), jnp.float32),
        scratch_types=[
            pltpu.VMEM((n_chunks, CHUNK), jnp.int32),
            pltpu.VMEM((CHUNK, emb), jnp.float32),
            pltpu.VMEM((CHUNK, emb), jnp.float32),
            pltpu.SemaphoreType.DMA,
            pltpu.SemaphoreType.DMA,
        ],
    )
    def k(table_hbm, idx_hbm, out_hbm, idx_v, row0, row1, g0, g1):
        wid = lax.axis_index("s") * NC + lax.axis_index("c")
        base = wid * bpw
        pltpu.sync_copy(idx_hbm.at[wid], idx_v)

        pltpu.async_copy(table_hbm.at[idx_v.at[0]], row0, g0)
        pltpu.async_copy(table_hbm.at[idx_v.at[1]], row1, g1)

        def body(p, _):
            c0 = 2 * p
            pltpu.make_async_copy(table_hbm.at[idx_v.at[c0]], row0, g0).wait()
            pltpu.sync_copy(row0, out_hbm.at[pl.ds(base + c0 * CHUNK, CHUNK)])
            pltpu.async_copy(table_hbm.at[idx_v.at[c0 + 2]], row0, g0)
            pltpu.make_async_copy(table_hbm.at[idx_v.at[c0 + 1]], row1, g1).wait()
            pltpu.sync_copy(row1, out_hbm.at[pl.ds(base + (c0 + 1) * CHUNK, CHUNK)])
            pltpu.async_copy(table_hbm.at[idx_v.at[c0 + 3]], row1, g1)
            return _

        lax.fori_loop(0, n_pairs, body, None)

        cl = n_chunks - 2
        pltpu.make_async_copy(table_hbm.at[idx_v.at[cl]], row0, g0).wait()
        pltpu.sync_copy(row0, out_hbm.at[pl.ds(base + cl * CHUNK, CHUNK)])
        pltpu.make_async_copy(table_hbm.at[idx_v.at[cl + 1]], row1, g1).wait()
        pltpu.sync_copy(row1, out_hbm.at[pl.ds(base + (cl + 1) * CHUNK, CHUNK)])

    out = k(table, idx3)
    return out.reshape(batch, seq, emb)


def kernel(input_tensor, embedding_table):
    batch, seq = input_tensor.shape
    _, emb = embedding_table.shape
    idx3 = input_tensor.astype(jnp.int32).reshape(NW, -1, CHUNK)
    return _embedding_gather(idx3, embedding_table, batch=batch, seq=seq, emb=emb)
